# baseline (speedup 1.0000x reference)
import functools

import jax
import jax.numpy as jnp
from jax import lax
from jax.experimental import pallas as pl
from jax.experimental.pallas import tpu as pltpu
from jax.experimental.pallas import tpu_sc as plsc

N = 10000
E = 320000
D = 128
B = 64
NC = 2
NS = 16
NW = NC * NS
N1 = 10240
STRIPE = N1 // NS
CHUNK = 80
EPT = E // NW
NCH = EPT // CHUNK
GRPC = 48
NTRI = NCH // 3
RB = 1024
NRB = N1 // RB

_mesh = plsc.VectorSubcoreMesh(core_axis_name="c", subcore_axis_name="s")


@functools.partial(
    pl.kernel,
    out_type=jax.ShapeDtypeStruct((NC, N1), jnp.float32),
    mesh=_mesh,
    scratch_types=[
        pltpu.VMEM((EPT,), jnp.int32),
        pltpu.VMEM((CHUNK,), jnp.float32),
        pltpu.VMEM((STRIPE,), jnp.float32),
        pltpu.VMEM_SHARED((N1,), jnp.float32),
    ],
)
def _deg_kernel(edge_hbm, deg_hbm, idxb, ones_v, zb, acc):
    c = lax.axis_index("c")
    sid = lax.axis_index("s")
    g = c * NS + sid
    base = sid * STRIPE

    def fill(k, _):
        zb[pl.ds(k * 16, 16)] = jnp.zeros((16,), jnp.float32)
        return 0

    lax.fori_loop(0, STRIPE // 16, fill, 0)

    def fill1(k, _):
        ones_v[pl.ds(k * 16, 16)] = jnp.ones((16,), jnp.float32)
        return 0

    lax.fori_loop(0, CHUNK // 16, fill1, 0)
    pltpu.sync_copy(zb, acc.at[pl.ds(base, STRIPE)])
    plsc.subcore_barrier()

    pltpu.sync_copy(edge_hbm.at[1, g], idxb)

    def body(ch, _):
        pltpu.sync_copy(ones_v, acc.at[idxb.at[pl.ds(ch * CHUNK, CHUNK)]], add=True)
        return 0

    lax.fori_loop(0, NCH, body, 0)
    plsc.subcore_barrier()
    pltpu.sync_copy(acc.at[pl.ds(base, STRIPE)], deg_hbm.at[c, pl.ds(base, STRIPE)])


@functools.partial(
    pl.kernel,
    out_type=jax.ShapeDtypeStruct((NC, N1, D), jnp.float32),
    mesh=_mesh,
    scratch_types=[
        pltpu.VMEM((EPT,), jnp.int32),
        pltpu.VMEM((GRPC * CHUNK,), jnp.int32),
        pltpu.VMEM((CHUNK, D), jnp.float32),
        pltpu.VMEM((CHUNK, D), jnp.float32),
        pltpu.VMEM((CHUNK, D), jnp.float32),
        pltpu.VMEM_SHARED((N1, D), jnp.float32),
        pltpu.SemaphoreType.DMA,
        pltpu.SemaphoreType.DMA,
        pltpu.SemaphoreType.DMA,
    ],
)
def _agg_kernel(edge_hbm, dstg_hbm, tbl_hbm, out_hbm, idx_s, idx_d, rows0,
                rows1, rows2, acc, sem0, sem1, sem2):
    c = lax.axis_index("c")
    sid = lax.axis_index("s")
    g = c * NS + sid
    base = sid * STRIPE

    def zr(r, _):
        def zc(k, _):
            rows0[r, pl.ds(k * 16, 16)] = jnp.zeros((16,), jnp.float32)
            return 0

        return lax.fori_loop(0, D // 16, zc, 0)

    lax.fori_loop(0, CHUNK, zr, 0)

    def zs(k, _):
        pltpu.sync_copy(rows0, acc.at[pl.ds(base + k * CHUNK, CHUNK)])
        return 0

    lax.fori_loop(0, STRIPE // CHUNK, zs, 0)
    plsc.subcore_barrier()

    pltpu.sync_copy(edge_hbm.at[0, g], idx_s)

    def sidx(ch):
        return idx_s.at[pl.ds(ch * CHUNK, CHUNK)]

    rows = (rows0, rows1, rows2)
    sems = (sem0, sem1, sem2)
    for k in range(3):
        pltpu.async_copy(tbl_hbm.at[sidx(k)], rows[k], sems[k])

    def make_triple(grp):
        def triple(t, _):
            for k in range(3):
                ch = 3 * t + k
                rel = ch - grp * GRPC
                pltpu.make_async_copy(tbl_hbm.at[sidx(ch)], rows[k], sems[k]).wait()
                pltpu.sync_copy(rows[k],
                                acc.at[idx_d.at[pl.ds(rel * CHUNK, CHUNK)]],
                                add=True)

                @pl.when(ch + 3 < NCH)
                def _():
                    pltpu.async_copy(tbl_hbm.at[sidx(ch + 3)], rows[k], sems[k])

            return 0

        return triple

    for grp in range(3):
        t0 = grp * GRPC // 3
        t1 = t0 + min(GRPC, 3 * (NTRI - t0)) // 3
        pltpu.sync_copy(dstg_hbm.at[g, pl.ds(grp * GRPC * CHUNK, GRPC * CHUNK)],
                        idx_d)
        lax.fori_loop(t0, t1, make_triple(grp), 0)

    for ch in range(3 * NTRI, NCH):
        k = ch % 3
        rel = ch - 2 * GRPC
        pltpu.make_async_copy(tbl_hbm.at[sidx(ch)], rows[k], sems[k]).wait()
        pltpu.sync_copy(rows[k], acc.at[idx_d.at[pl.ds(rel * CHUNK, CHUNK)]],
                        add=True)
    plsc.subcore_barrier()

    def wo(k, _):
        r0 = base + k * 128
        pltpu.sync_copy(acc.at[pl.ds(r0, 128)], out_hbm.at[c, pl.ds(r0, 128)])
        return 0

    lax.fori_loop(0, STRIPE // 128, wo, 0)


def _dis_of(deg_ref):
    deg = deg_ref[0] + deg_ref[1]
    return jnp.where(deg > 0, lax.rsqrt(jnp.maximum(deg, 1e-12)), 0.0)


def _mm_body(x_ref, w_ref, o_ref):
    o_ref[...] = jnp.dot(x_ref[...], w_ref[...], preferred_element_type=jnp.float32)


def _scale_body(xw_ref, deg_ref, o_ref):
    dis = _dis_of(deg_ref)
    o_ref[...] = xw_ref[...] * dis[:, None]


def _mid_body(acc_ref, s_ref, deg_ref, b1_ref, w2_ref, o_ref):
    dis = _dis_of(deg_ref)
    hp = acc_ref[0] + acc_ref[1] + s_ref[...]
    h = jnp.maximum(hp * dis[:, None] + b1_ref[...], 0.0)
    o_ref[...] = jnp.dot(h, w2_ref[...], preferred_element_type=jnp.float32) * dis[:, None]


def _head_body(acc_ref, s_ref, deg_ref, b2_ref, bt_ref, wl1_ref, bl1_ref,
               wl2_ref, bl2_ref, o_ref, pooled, cnt):
    i = pl.program_id(0)

    @pl.when(i == 0)
    def _():
        pooled[...] = jnp.zeros_like(pooled)
        cnt[...] = jnp.zeros_like(cnt)

    dis = _dis_of(deg_ref)
    h2 = (acc_ref[0] + acc_ref[1] + s_ref[...]) * dis[:, None] + b2_ref[...]
    bt = bt_ref[0, 0]
    ohT = (lax.broadcasted_iota(jnp.int32, (B, RB), 0) == bt[None, :]).astype(jnp.float32)
    pooled[...] += jnp.dot(ohT, h2, preferred_element_type=jnp.float32)
    cnt[...] += jnp.sum(ohT, axis=1, keepdims=True)

    @pl.when(i == NRB - 1)
    def _():
        pm = pooled[...] / jnp.maximum(cnt[...], 1.0)
        z = jnp.maximum(
            jnp.dot(pm, wl1_ref[...], preferred_element_type=jnp.float32)
            + bl1_ref[...], 0.0)
        logit = jnp.sum(z * wl2_ref[...], axis=1, keepdims=True) + bl2_ref[...]
        o_ref[...] = 1.0 / (1.0 + jnp.exp(-logit))


def kernel(x, edge_index, batch, W1, b1, W2, b2, Wl1, bl1, Wl2, bl2):
    f32 = jnp.float32
    edge3 = edge_index.reshape(2, NW, EPT)
    dstg = jnp.pad(edge_index[1].reshape(NW, EPT),
                   ((0, 0), (0, 3 * GRPC * CHUNK - EPT)),
                   constant_values=N1 - 1)
    xp = jnp.pad(x, ((0, N1 - N), (0, 0)))
    btp = jnp.pad(batch, (0, N1 - N), constant_values=B).reshape(NRB, 1, RB)
    b1r = b1.reshape(1, D)
    b2r = b2.reshape(1, D)
    bl1r = bl1.reshape(1, 64)
    wl2r = Wl2.reshape(1, 64)
    bl2r = bl2.reshape(1, 1)

    degp = _deg_kernel(edge3)

    xw = pl.pallas_call(
        _mm_body,
        grid=(NRB,),
        in_specs=[
            pl.BlockSpec((RB, D), lambda i: (i, 0)),
            pl.BlockSpec((D, D), lambda i: (0, 0)),
        ],
        out_specs=pl.BlockSpec((RB, D), lambda i: (i, 0)),
        out_shape=jax.ShapeDtypeStruct((N1, D), f32),
    )(xp, W1)

    s1 = pl.pallas_call(
        _scale_body,
        grid=(NRB,),
        in_specs=[
            pl.BlockSpec((RB, D), lambda i: (i, 0)),
            pl.BlockSpec((NC, RB), lambda i: (0, i)),
        ],
        out_specs=pl.BlockSpec((RB, D), lambda i: (i, 0)),
        out_shape=jax.ShapeDtypeStruct((N1, D), f32),
    )(xw, degp)

    acc1 = _agg_kernel(edge3, dstg, s1)

    s2 = pl.pallas_call(
        _mid_body,
        grid=(NRB,),
        in_specs=[
            pl.BlockSpec((NC, RB, D), lambda i: (0, i, 0)),
            pl.BlockSpec((RB, D), lambda i: (i, 0)),
            pl.BlockSpec((NC, RB), lambda i: (0, i)),
            pl.BlockSpec((1, D), lambda i: (0, 0)),
            pl.BlockSpec((D, D), lambda i: (0, 0)),
        ],
        out_specs=pl.BlockSpec((RB, D), lambda i: (i, 0)),
        out_shape=jax.ShapeDtypeStruct((N1, D), f32),
    )(acc1, s1, degp, b1r, W2)

    acc2 = _agg_kernel(edge3, dstg, s2)

    out = pl.pallas_call(
        _head_body,
        grid=(NRB,),
        in_specs=[
            pl.BlockSpec((NC, RB, D), lambda i: (0, i, 0)),
            pl.BlockSpec((RB, D), lambda i: (i, 0)),
            pl.BlockSpec((NC, RB), lambda i: (0, i)),
            pl.BlockSpec((1, D), lambda i: (0, 0)),
            pl.BlockSpec((1, 1, RB), lambda i: (i, 0, 0)),
            pl.BlockSpec((D, 64), lambda i: (0, 0)),
            pl.BlockSpec((1, 64), lambda i: (0, 0)),
            pl.BlockSpec((1, 64), lambda i: (0, 0)),
            pl.BlockSpec((1, 1), lambda i: (0, 0)),
        ],
        out_specs=pl.BlockSpec((B, 1), lambda i: (0, 0)),
        out_shape=jax.ShapeDtypeStruct((B, 1), f32),
        scratch_shapes=[
            pltpu.VMEM((B, D), f32),
            pltpu.VMEM((B, 1), f32),
        ],
    )(acc2, s2, degp, b2r, btp, Wl1, bl1r, wl2r, bl2r)

    return out


# speedup vs baseline: 1.0361x; 1.0361x over previous
import functools

import jax
import jax.numpy as jnp
from jax import lax
from jax.experimental import pallas as pl
from jax.experimental.pallas import tpu as pltpu
from jax.experimental.pallas import tpu_sc as plsc

N = 10000
E = 320000
D = 128
B = 64
NC = 2
NS = 16
NW = NC * NS
N1 = 10240
STRIPE = N1 // NS
CHUNK = 80
EPT = E // NW
NCH = EPT // CHUNK
GRPC = 48
NTRI = NCH // 3
RB = 1024
NRB = N1 // RB

_mesh = plsc.VectorSubcoreMesh(core_axis_name="c", subcore_axis_name="s")


@functools.partial(
    pl.kernel,
    out_type=jax.ShapeDtypeStruct((NC, N1), jnp.float32),
    mesh=_mesh,
    scratch_types=[
        pltpu.VMEM((EPT,), jnp.int32),
        pltpu.VMEM((CHUNK,), jnp.float32),
        pltpu.VMEM((STRIPE,), jnp.float32),
        pltpu.VMEM_SHARED((N1,), jnp.float32),
    ],
)
def _deg_kernel(edge_hbm, deg_hbm, idxb, ones_v, zb, acc):
    c = lax.axis_index("c")
    sid = lax.axis_index("s")
    g = c * NS + sid
    base = sid * STRIPE

    def fill(k, _):
        zb[pl.ds(k * 16, 16)] = jnp.zeros((16,), jnp.float32)
        return 0

    lax.fori_loop(0, STRIPE // 16, fill, 0)

    def fill1(k, _):
        ones_v[pl.ds(k * 16, 16)] = jnp.ones((16,), jnp.float32)
        return 0

    lax.fori_loop(0, CHUNK // 16, fill1, 0)
    pltpu.sync_copy(zb, acc.at[pl.ds(base, STRIPE)])
    plsc.subcore_barrier()

    pltpu.sync_copy(edge_hbm.at[1, g], idxb)

    def body(ch, _):
        pltpu.sync_copy(ones_v, acc.at[idxb.at[pl.ds(ch * CHUNK, CHUNK)]], add=True)
        return 0

    lax.fori_loop(0, NCH, body, 0)
    plsc.subcore_barrier()
    pltpu.sync_copy(acc.at[pl.ds(base, STRIPE)], deg_hbm.at[c, pl.ds(base, STRIPE)])


@functools.partial(
    pl.kernel,
    out_type=jax.ShapeDtypeStruct((NC, N1, D), jnp.float32),
    mesh=_mesh,
    scratch_types=[
        pltpu.VMEM((EPT,), jnp.int32),
        pltpu.VMEM((GRPC * CHUNK,), jnp.int32),
        pltpu.VMEM((CHUNK, D), jnp.float32),
        pltpu.VMEM((CHUNK, D), jnp.float32),
        pltpu.VMEM((CHUNK, D), jnp.float32),
        pltpu.VMEM_SHARED((N1, D), jnp.float32),
        pltpu.SemaphoreType.DMA,
        pltpu.SemaphoreType.DMA,
        pltpu.SemaphoreType.DMA,
        pltpu.SemaphoreType.DMA,
        pltpu.SemaphoreType.DMA,
        pltpu.SemaphoreType.DMA,
    ],
)
def _agg_kernel(edge_hbm, dstg_hbm, tbl_hbm, out_hbm, idx_s, idx_d, rows0,
                rows1, rows2, acc, sem0, sem1, sem2, ssm0, ssm1, ssm2):
    c = lax.axis_index("c")
    sid = lax.axis_index("s")
    g = c * NS + sid
    base = sid * STRIPE

    def zr(r, _):
        def zc(k, _):
            rows0[r, pl.ds(k * 16, 16)] = jnp.zeros((16,), jnp.float32)
            return 0

        return lax.fori_loop(0, D // 16, zc, 0)

    lax.fori_loop(0, CHUNK, zr, 0)

    def zs(k, _):
        pltpu.sync_copy(rows0, acc.at[pl.ds(base + k * CHUNK, CHUNK)])
        return 0

    lax.fori_loop(0, STRIPE // CHUNK, zs, 0)
    plsc.subcore_barrier()

    pltpu.sync_copy(edge_hbm.at[0, g], idx_s)

    def sidx(ch):
        return idx_s.at[pl.ds(ch * CHUNK, CHUNK)]

    rows = (rows0, rows1, rows2)
    sems = (sem0, sem1, sem2)
    ssems = (ssm0, ssm1, ssm2)
    for k in range(3):
        pltpu.async_copy(tbl_hbm.at[sidx(k)], rows[k], sems[k])

    def make_triple(grp):
        def triple(t, _):
            for k in range(3):
                ch = 3 * t + k
                rel = ch - grp * GRPC
                dref = acc.at[idx_d.at[pl.ds(rel * CHUNK, CHUNK)]]
                pltpu.make_async_copy(tbl_hbm.at[sidx(ch)], rows[k], sems[k]).wait()
                pltpu.async_copy(rows[k], dref, ssems[k], add=True)

                @pl.when(ch + 3 < NCH)
                def _():
                    pltpu.make_async_copy(rows[k], dref, ssems[k]).wait()
                    pltpu.async_copy(tbl_hbm.at[sidx(ch + 3)], rows[k], sems[k])

            return 0

        return triple

    for grp in range(3):
        t0 = grp * GRPC // 3
        t1 = t0 + min(GRPC, 3 * (NTRI - t0)) // 3
        pltpu.sync_copy(dstg_hbm.at[g, pl.ds(grp * GRPC * CHUNK, GRPC * CHUNK)],
                        idx_d)
        lax.fori_loop(t0, t1, make_triple(grp), 0)

    for ch in range(3 * NTRI, NCH):
        k = ch % 3
        rel = ch - 2 * GRPC
        pltpu.make_async_copy(tbl_hbm.at[sidx(ch)], rows[k], sems[k]).wait()
        pltpu.async_copy(rows[k], acc.at[idx_d.at[pl.ds(rel * CHUNK, CHUNK)]],
                         ssems[k], add=True)
    for ch in range(NCH - 3, NCH):
        k = ch % 3
        rel = ch - 2 * GRPC
        pltpu.make_async_copy(rows[k], acc.at[idx_d.at[pl.ds(rel * CHUNK, CHUNK)]],
                              ssems[k]).wait()
    plsc.subcore_barrier()

    def wo(k, _):
        r0 = base + k * 128
        pltpu.sync_copy(acc.at[pl.ds(r0, 128)], out_hbm.at[c, pl.ds(r0, 128)])
        return 0

    lax.fori_loop(0, STRIPE // 128, wo, 0)


def _dis_of(deg_ref):
    deg = deg_ref[0] + deg_ref[1]
    return jnp.where(deg > 0, lax.rsqrt(jnp.maximum(deg, 1e-12)), 0.0)


def _scale_body(x_ref, deg_ref, w_ref, o_ref):
    dis = _dis_of(deg_ref)
    xw = jnp.dot(x_ref[...], w_ref[...], preferred_element_type=jnp.float32)
    o_ref[...] = xw * dis[:, None]


def _mid_body(acc_ref, s_ref, deg_ref, b1_ref, w2_ref, o_ref):
    dis = _dis_of(deg_ref)
    hp = acc_ref[0] + acc_ref[1] + s_ref[...]
    h = jnp.maximum(hp * dis[:, None] + b1_ref[...], 0.0)
    o_ref[...] = jnp.dot(h, w2_ref[...], preferred_element_type=jnp.float32) * dis[:, None]


def _head_body(acc_ref, s_ref, deg_ref, b2_ref, bt_ref, wl1_ref, bl1_ref,
               wl2_ref, bl2_ref, o_ref, pooled, cnt):
    i = pl.program_id(0)

    @pl.when(i == 0)
    def _():
        pooled[...] = jnp.zeros_like(pooled)
        cnt[...] = jnp.zeros_like(cnt)

    dis = _dis_of(deg_ref)
    h2 = (acc_ref[0] + acc_ref[1] + s_ref[...]) * dis[:, None] + b2_ref[...]
    bt = bt_ref[0, 0]
    ohT = (lax.broadcasted_iota(jnp.int32, (B, RB), 0) == bt[None, :]).astype(jnp.float32)
    pooled[...] += jnp.dot(ohT, h2, preferred_element_type=jnp.float32)
    cnt[...] += jnp.sum(ohT, axis=1, keepdims=True)

    @pl.when(i == NRB - 1)
    def _():
        pm = pooled[...] / jnp.maximum(cnt[...], 1.0)
        z = jnp.maximum(
            jnp.dot(pm, wl1_ref[...], preferred_element_type=jnp.float32)
            + bl1_ref[...], 0.0)
        logit = jnp.sum(z * wl2_ref[...], axis=1, keepdims=True) + bl2_ref[...]
        o_ref[...] = 1.0 / (1.0 + jnp.exp(-logit))


def kernel(x, edge_index, batch, W1, b1, W2, b2, Wl1, bl1, Wl2, bl2):
    f32 = jnp.float32
    edge3 = edge_index.reshape(2, NW, EPT)
    dstg = jnp.pad(edge_index[1].reshape(NW, EPT),
                   ((0, 0), (0, 3 * GRPC * CHUNK - EPT)),
                   constant_values=N1 - 1)
    xp = jnp.pad(x, ((0, N1 - N), (0, 0)))
    btp = jnp.pad(batch, (0, N1 - N), constant_values=B).reshape(NRB, 1, RB)
    b1r = b1.reshape(1, D)
    b2r = b2.reshape(1, D)
    bl1r = bl1.reshape(1, 64)
    wl2r = Wl2.reshape(1, 64)
    bl2r = bl2.reshape(1, 1)

    degp = _deg_kernel(edge3)

    s1 = pl.pallas_call(
        _scale_body,
        grid=(NRB,),
        in_specs=[
            pl.BlockSpec((RB, D), lambda i: (i, 0)),
            pl.BlockSpec((NC, RB), lambda i: (0, i)),
            pl.BlockSpec((D, D), lambda i: (0, 0)),
        ],
        out_specs=pl.BlockSpec((RB, D), lambda i: (i, 0)),
        out_shape=jax.ShapeDtypeStruct((N1, D), f32),
    )(xp, degp, W1)

    acc1 = _agg_kernel(edge3, dstg, s1)

    s2 = pl.pallas_call(
        _mid_body,
        grid=(NRB,),
        in_specs=[
            pl.BlockSpec((NC, RB, D), lambda i: (0, i, 0)),
            pl.BlockSpec((RB, D), lambda i: (i, 0)),
            pl.BlockSpec((NC, RB), lambda i: (0, i)),
            pl.BlockSpec((1, D), lambda i: (0, 0)),
            pl.BlockSpec((D, D), lambda i: (0, 0)),
        ],
        out_specs=pl.BlockSpec((RB, D), lambda i: (i, 0)),
        out_shape=jax.ShapeDtypeStruct((N1, D), f32),
    )(acc1, s1, degp, b1r, W2)

    acc2 = _agg_kernel(edge3, dstg, s2)

    out = pl.pallas_call(
        _head_body,
        grid=(NRB,),
        in_specs=[
            pl.BlockSpec((NC, RB, D), lambda i: (0, i, 0)),
            pl.BlockSpec((RB, D), lambda i: (i, 0)),
            pl.BlockSpec((NC, RB), lambda i: (0, i)),
            pl.BlockSpec((1, D), lambda i: (0, 0)),
            pl.BlockSpec((1, 1, RB), lambda i: (i, 0, 0)),
            pl.BlockSpec((D, 64), lambda i: (0, 0)),
            pl.BlockSpec((1, 64), lambda i: (0, 0)),
            pl.BlockSpec((1, 64), lambda i: (0, 0)),
            pl.BlockSpec((1, 1), lambda i: (0, 0)),
        ],
        out_specs=pl.BlockSpec((B, 1), lambda i: (0, 0)),
        out_shape=jax.ShapeDtypeStruct((B, 1), f32),
        scratch_shapes=[
            pltpu.VMEM((B, D), f32),
            pltpu.VMEM((B, 1), f32),
        ],
    )(acc2, s2, degp, b2r, btp, Wl1, bl1r, wl2r, bl2r)

    return out
